# baseline (speedup 1.0000x reference)
import functools

import jax
import jax.numpy as jnp
from jax import lax
from jax.experimental import pallas as pl
from jax.experimental.pallas import tpu as pltpu
from jax.experimental.pallas import tpu_sc as plsc

D = 64
S = 50
NBT = 128
NC, NS = 2, 16
NW = NC * NS
BTW = NBT // NW

_mesh = plsc.VectorSubcoreMesh(core_axis_name="c", subcore_axis_name="s")


@functools.partial(
    pl.kernel,
    out_type=jax.ShapeDtypeStruct((S, 8, NBT, 1024), jnp.float32),
    mesh=_mesh,
    scratch_types=[
        pltpu.VMEM((S, BTW, 128), jnp.int32),
        pltpu.VMEM((2, 128, D), jnp.float32),
        pltpu.VMEM((2, 8, 1024), jnp.float32),
        pltpu.SemaphoreType.DMA,
        pltpu.SemaphoreType.DMA,
        pltpu.SemaphoreType.DMA,
        pltpu.SemaphoreType.DMA,
    ],
    compiler_params=pltpu.CompilerParams(
        use_tc_tiling_on_sc=False, needs_layout_passes=False),
)
def _emb_lookup(idx_hbm, table_hbm, out_hbm, idx_v, gbuf, tbuf,
                sg0, sg1, sw0, sw1):
    wid = lax.axis_index("s") * NC + lax.axis_index("c")
    bt0 = wid * BTW
    pltpu.sync_copy(idx_hbm.at[:, pl.ds(bt0, BTW)], idx_v)
    sgs = (sg0, sg1)
    sws = (sw0, sw1)

    iota = lax.iota(jnp.int32, 16)
    rows = [g * 16 + iota for g in range(8)]

    def fire_gather(s, bl):
        par = bl % 2
        pltpu.async_copy(table_hbm.at[idx_v.at[s, bl]], gbuf.at[par], sgs[par])

    def wait_gather(par):
        pltpu.make_async_copy(
            table_hbm.at[idx_v.at[0, 0]], gbuf.at[par], sgs[par]).wait()

    def transpose(par):
        @pl.loop(0, D)
        def _d(d):
            dt = d // 8
            off = (d % 8) * 128
            dcol = jnp.zeros((16,), jnp.int32) + d
            for g in range(8):
                x = plsc.load_gather(gbuf.at[par], [rows[g], dcol])
                tbuf[par, dt, pl.ds(off + g * 16, 16)] = x

    def fire_write(s, bl):
        par = bl % 2
        pltpu.async_copy(tbuf.at[par], out_hbm.at[s, :, bt0 + bl], sws[par])

    def wait_write(par):
        pltpu.make_async_copy(
            tbuf.at[par], out_hbm.at[0, :, 0], sws[par]).wait()

    def step(s, bl, fire_next=True, wait_wr=True):
        par = bl % 2
        wait_gather(par)
        if fire_next:
            if bl < BTW - 1:
                fire_gather(s, bl + 1)
            else:
                fire_gather(s + 1, 0)
        if wait_wr:
            wait_write(par)
        transpose(par)
        fire_write(s, bl)

    fire_gather(0, 0)
    step(0, 0, wait_wr=False)
    step(0, 1, wait_wr=False)
    step(0, 2)
    step(0, 3)

    @pl.loop(1, S - 1)
    def _steady(s):
        for bl in range(BTW):
            step(s, bl)

    step(S - 1, 0)
    step(S - 1, 1)
    step(S - 1, 2)
    step(S - 1, 3, fire_next=False)
    wait_write(0)
    wait_write(1)


def kernel(input, weight):
    idx3 = input.T.reshape(S, NBT, 128).astype(jnp.int32)
    out5 = _emb_lookup(idx3, weight).reshape(S, 8, NBT, 8, 128)
    return out5.transpose(2, 4, 0, 1, 3).reshape(16384, S, D)


# speedup vs baseline: 1.3846x; 1.3846x over previous
import functools

import jax
import jax.numpy as jnp
from jax import lax
from jax.experimental import pallas as pl
from jax.experimental.pallas import tpu as pltpu
from jax.experimental.pallas import tpu_sc as plsc

D = 64
S = 50
NBT = 128
NC, NS = 2, 16
NW = NC * NS
BTW = NBT // NW
NP = S * BTW

_mesh = plsc.VectorSubcoreMesh(core_axis_name="c", subcore_axis_name="s")


@functools.partial(
    pl.kernel,
    out_type=jax.ShapeDtypeStruct((S, 8, NBT, 1024), jnp.float32),
    mesh=_mesh,
    scratch_types=[
        pltpu.VMEM((S, BTW, 128), jnp.int32),
        pltpu.VMEM((2, 128, D), jnp.float32),
        pltpu.VMEM((2, 8, 1024), jnp.float32),
        pltpu.SemaphoreType.DMA((2,)),
        pltpu.SemaphoreType.DMA((2,)),
    ],
    compiler_params=pltpu.CompilerParams(
        use_tc_tiling_on_sc=False, needs_layout_passes=False),
)
def _emb_lookup(idx_hbm, table_hbm, out_hbm, idx_v, gbuf, tbuf, sg, sw):
    wid = lax.axis_index("s") * NC + lax.axis_index("c")
    bt0 = wid * BTW
    pltpu.sync_copy(idx_hbm.at[:, pl.ds(bt0, BTW)], idx_v)

    iota = lax.iota(jnp.int32, 16)
    rows = [g * 16 + iota for g in range(8)]

    def fire_gather(p, par):
        s = p // BTW
        bl = lax.rem(p, BTW)
        pltpu.async_copy(
            table_hbm.at[idx_v.at[s, bl]], gbuf.at[par], sg.at[par])

    def transpose(par):
        @plsc.parallel_loop(0, D, unroll=8)
        def _d(d):
            dcol = jnp.zeros((16,), jnp.int32) + d
            dt = d // 8
            base = lax.rem(d, 8) * 128
            for g in range(8):
                x = plsc.load_gather(gbuf.at[par], [rows[g], dcol])
                tbuf[par, dt, pl.ds(base + g * 16, 16)] = x

    @pl.loop(0, NP)
    def _pair(p):
        par = lax.rem(p, 2)
        s = p // BTW
        bl = lax.rem(p, BTW)

        @pl.when(p == 0)
        def _prime():
            fire_gather(p, par)

        pltpu.make_async_copy(
            table_hbm.at[idx_v.at[0, 0]], gbuf.at[par], sg.at[par]).wait()

        @pl.when(p < NP - 1)
        def _next():
            fire_gather(p + 1, 1 - par)

        @pl.when(p >= 2)
        def _drain():
            pltpu.make_async_copy(
                tbuf.at[par], out_hbm.at[0, :, 0], sw.at[par]).wait()

        transpose(par)
        pltpu.async_copy(tbuf.at[par], out_hbm.at[s, :, bt0 + bl], sw.at[par])

    pltpu.make_async_copy(tbuf.at[0], out_hbm.at[0, :, 0], sw.at[0]).wait()
    pltpu.make_async_copy(tbuf.at[1], out_hbm.at[0, :, 0], sw.at[1]).wait()


def kernel(input, weight):
    idx3 = input.T.reshape(S, NBT, 128).astype(jnp.int32)
    out5 = _emb_lookup(idx3, weight).reshape(S, 8, NBT, 8, 128)
    return out5.transpose(2, 4, 0, 1, 3).reshape(16384, S, D)


# speedup vs baseline: 2.1526x; 1.5547x over previous
import functools

import jax
import jax.numpy as jnp
from jax import lax
from jax.experimental import pallas as pl
from jax.experimental.pallas import tpu as pltpu
from jax.experimental.pallas import tpu_sc as plsc

D = 64
S = 50
NBT = 128
NC, NS = 2, 16
NW = NC * NS
BTW = NBT // NW
NP = S * BTW

_mesh = plsc.VectorSubcoreMesh(core_axis_name="c", subcore_axis_name="s")


@functools.partial(
    pl.kernel,
    out_type=jax.ShapeDtypeStruct((S, 4, 2, NBT, 8, 128), jnp.float32),
    mesh=_mesh,
    scratch_types=[
        pltpu.VMEM((S, BTW, 128), jnp.int32),
        pltpu.VMEM((2, 128, D), jnp.float32),
        pltpu.VMEM((2, 4, 2, 8, 131), jnp.float32),
        pltpu.SemaphoreType.DMA((2,)),
        pltpu.SemaphoreType.DMA((2,)),
    ],
    compiler_params=pltpu.CompilerParams(
        use_tc_tiling_on_sc=False, needs_layout_passes=False),
)
def _emb_lookup(idx_hbm, table_hbm, out_hbm, idx_v, gbuf, tbuf, sg, sw):
    wid = lax.axis_index("s") * NC + lax.axis_index("c")
    bt0 = wid * BTW
    pltpu.sync_copy(idx_hbm.at[:, pl.ds(bt0, BTW)], idx_v)

    iota = lax.iota(jnp.int32, 16)
    dtlo_c = iota // 8
    dr_c = lax.rem(iota, 8)

    def fire_gather(p, par):
        s = p // BTW
        bl = lax.rem(p, BTW)
        pltpu.async_copy(
            table_hbm.at[idx_v.at[s, bl]], gbuf.at[par], sg.at[par])

    def transpose(par):
        @plsc.parallel_loop(0, 128, unroll=8)
        def _bc(bc):
            bcv = jnp.zeros((16,), jnp.int32) + bc
            for g in range(4):
                x = gbuf[par, bc, pl.ds(g * 16, 16)]
                plsc.store_scatter(tbuf.at[par, g], [dtlo_c, dr_c, bcv], x)

    @pl.loop(0, NP)
    def _pair(p):
        par = lax.rem(p, 2)
        s = p // BTW
        bl = lax.rem(p, BTW)

        @pl.when(p == 0)
        def _prime():
            fire_gather(p, par)

        pltpu.make_async_copy(
            table_hbm.at[idx_v.at[0, 0]], gbuf.at[par], sg.at[par]).wait()

        @pl.when(p < NP - 1)
        def _next():
            fire_gather(p + 1, 1 - par)

        @pl.when(p >= 2)
        def _drain():
            pltpu.make_async_copy(
                tbuf.at[par, :, :, :, pl.ds(0, 128)],
                out_hbm.at[0, :, :, 0], sw.at[par]).wait()

        transpose(par)
        pltpu.async_copy(
            tbuf.at[par, :, :, :, pl.ds(0, 128)],
            out_hbm.at[s, :, :, bt0 + bl], sw.at[par])

    for q in range(2):
        pltpu.make_async_copy(
            tbuf.at[q, :, :, :, pl.ds(0, 128)],
            out_hbm.at[0, :, :, 0], sw.at[q]).wait()


def kernel(input, weight):
    idx3 = input.T.reshape(S, NBT, 128).astype(jnp.int32)
    out5 = _emb_lookup(idx3, weight).reshape(S, 8, NBT, 8, 128)
    return out5.transpose(2, 4, 0, 1, 3).reshape(16384, S, D)


# speedup vs baseline: 3.3003x; 1.5332x over previous
import functools

import jax
import jax.numpy as jnp
from jax import lax
from jax.experimental import pallas as pl
from jax.experimental.pallas import tpu as pltpu
from jax.experimental.pallas import tpu_sc as plsc

D = 64
S = 50
NBT = 128
NC, NS = 2, 16
NW = NC * NS
BTW = NBT // NW
NP = S * BTW

_mesh = plsc.VectorSubcoreMesh(core_axis_name="c", subcore_axis_name="s")

NVT = 7813
NVT_W = 245


@functools.partial(
    pl.kernel,
    out_type=jax.ShapeDtypeStruct((500000, 128), jnp.float32),
    mesh=_mesh,
    scratch_types=[
        pltpu.VMEM((2, D, 128), jnp.float32),
        pltpu.VMEM((2, D, 128), jnp.float32),
        pltpu.SemaphoreType.DMA((2,)),
        pltpu.SemaphoreType.DMA((2,)),
    ],
    compiler_params=pltpu.CompilerParams(
        use_tc_tiling_on_sc=True, needs_layout_passes=False),
)
def _format_table(wt_hbm, tail_hbm, w2_hbm, cbuf, pbuf, sv, sw):
    wid = lax.axis_index("s") * NC + lax.axis_index("c")
    iota = lax.iota(jnp.int32, 16)
    dvecs = [D0 + iota for D0 in range(0, D, 16)]

    def fire_slab(i, buf):
        vt = wid + i * NW

        @pl.when(vt < NVT - 1)
        def _body():
            pltpu.async_copy(
                wt_hbm.at[:, pl.ds(vt * 128, 128)], cbuf.at[buf], sv.at[buf])

        @pl.when(vt == NVT - 1)
        def _tail():
            pltpu.async_copy(tail_hbm, cbuf.at[buf], sv.at[buf])

    def transpose(buf):
        @plsc.parallel_loop(0, 16, unroll=2)
        def _j(j):
            perm = lax.rem(iota + j, 16)
            rbase = perm // 2
            cvecs = [lax.rem(perm, 2) * 64 + dv for dv in dvecs]
            for di in range(4):
                for V0 in range(0, 128, 16):
                    x = plsc.load_gather(cbuf.at[buf], [dvecs[di], V0 + perm])
                    plsc.store_scatter(
                        pbuf.at[buf], [V0 // 2 + rbase, cvecs[di]], x)

    def fire_write(vt, buf):
        @pl.when(vt < NVT - 1)
        def _full():
            pltpu.async_copy(pbuf.at[buf],
                             w2_hbm.at[pl.ds(vt * 64, 64)], sw.at[buf])

        @pl.when(vt == NVT - 1)
        def _tail():
            pltpu.async_copy(pbuf.at[buf, pl.ds(0, 32)],
                             w2_hbm.at[pl.ds(vt * 64, 32)], sw.at[buf])

    def wait_write(vt, buf):
        @pl.when(vt < NVT - 1)
        def _full():
            pltpu.make_async_copy(
                pbuf.at[buf], w2_hbm.at[pl.ds(0, 64)], sw.at[buf]).wait()

        @pl.when(vt == NVT - 1)
        def _tail():
            pltpu.make_async_copy(
                pbuf.at[buf, pl.ds(0, 32)],
                w2_hbm.at[pl.ds(0, 32)], sw.at[buf]).wait()

    @pl.loop(0, NVT_W)
    def _slab(i):
        vt = wid + i * NW

        @pl.when(vt < NVT)
        def _active():
            buf = lax.rem(i, 2)

            @pl.when(i == 0)
            def _prime():
                fire_slab(i, buf)

            pltpu.make_async_copy(
                wt_hbm.at[:, pl.ds(0, 128)], cbuf.at[buf], sv.at[buf]).wait()

            @pl.when(wid + (i + 1) * NW < NVT)
            def _next():
                fire_slab(i + 1, 1 - buf)

            @pl.when(i >= 2)
            def _drain():
                wait_write(wid + (i - 2) * NW, buf)

            transpose(buf)
            fire_write(vt, buf)

    n_act = jnp.where(wid < NVT - (NVT_W - 1) * NW, NVT_W, NVT_W - 1)

    @pl.loop(0, 2)
    def _final(q):
        i_last = n_act - 1 - q
        wait_write(wid + i_last * NW, lax.rem(i_last, 2))


@functools.partial(
    pl.kernel,
    out_type=jax.ShapeDtypeStruct((S, 4, 2, NBT, 8, 128), jnp.float32),
    mesh=_mesh,
    scratch_types=[
        pltpu.VMEM((S, BTW, 128), jnp.int32),
        pltpu.VMEM((2, 128, D), jnp.float32),
        pltpu.VMEM((2, 4, 2, 8, 131), jnp.float32),
        pltpu.SemaphoreType.DMA((2,)),
        pltpu.SemaphoreType.DMA((2,)),
    ],
    compiler_params=pltpu.CompilerParams(
        use_tc_tiling_on_sc=False, needs_layout_passes=False),
)
def _emb_lookup(idx_hbm, table_hbm, out_hbm, idx_v, gbuf, tbuf, sg, sw):
    wid = lax.axis_index("s") * NC + lax.axis_index("c")
    bt0 = wid * BTW
    pltpu.sync_copy(idx_hbm.at[:, pl.ds(bt0, BTW)], idx_v)

    iota = lax.iota(jnp.int32, 16)
    dtlo_c = iota // 8
    dr_c = lax.rem(iota, 8)

    def fire_gather(p, par):
        s = p // BTW
        bl = lax.rem(p, BTW)
        pltpu.async_copy(
            table_hbm.at[idx_v.at[s, bl]], gbuf.at[par], sg.at[par])

    def transpose(par):
        @plsc.parallel_loop(0, 128, unroll=8)
        def _bc(bc):
            bcv = jnp.zeros((16,), jnp.int32) + bc
            for g in range(4):
                x = gbuf[par, bc, pl.ds(g * 16, 16)]
                plsc.store_scatter(tbuf.at[par, g], [dtlo_c, dr_c, bcv], x)

    @pl.loop(0, NP)
    def _pair(p):
        par = lax.rem(p, 2)
        s = p // BTW
        bl = lax.rem(p, BTW)

        @pl.when(p == 0)
        def _prime():
            fire_gather(p, par)

        pltpu.make_async_copy(
            table_hbm.at[idx_v.at[0, 0]], gbuf.at[par], sg.at[par]).wait()

        @pl.when(p < NP - 1)
        def _next():
            fire_gather(p + 1, 1 - par)

        @pl.when(p >= 2)
        def _drain():
            pltpu.make_async_copy(
                tbuf.at[par, :, :, :, pl.ds(0, 128)],
                out_hbm.at[0, :, :, 0], sw.at[par]).wait()

        transpose(par)
        pltpu.async_copy(
            tbuf.at[par, :, :, :, pl.ds(0, 128)],
            out_hbm.at[s, :, :, bt0 + bl], sw.at[par])

    for q in range(2):
        pltpu.make_async_copy(
            tbuf.at[q, :, :, :, pl.ds(0, 128)],
            out_hbm.at[0, :, :, 0], sw.at[q]).wait()


def kernel(input, weight):
    idx3 = input.T.reshape(S, NBT, 128).astype(jnp.int32)
    wt = weight.T
    tailp = jnp.pad(wt[:, (NVT - 1) * 128:], ((0, 0), (0, 64)))
    table = _format_table(wt, tailp).reshape(1000000, D)
    out5 = _emb_lookup(idx3, table).reshape(S, 8, NBT, 8, 128)
    return out5.transpose(2, 4, 0, 1, 3).reshape(16384, S, D)


# speedup vs baseline: 3.3011x; 1.0002x over previous
import functools

import jax
import jax.numpy as jnp
from jax import lax
from jax.experimental import pallas as pl
from jax.experimental.pallas import tpu as pltpu
from jax.experimental.pallas import tpu_sc as plsc

D = 64
S = 50
NBT = 128
NC, NS = 2, 16
NW = NC * NS
BTW = NBT // NW
NP = S * BTW

_mesh = plsc.VectorSubcoreMesh(core_axis_name="c", subcore_axis_name="s")

NVT = 7813
NVT_W = 245


@functools.partial(
    pl.kernel,
    out_type=jax.ShapeDtypeStruct((500000, 128), jnp.float32),
    mesh=_mesh,
    scratch_types=[
        pltpu.VMEM((2, D, 128), jnp.float32),
        pltpu.VMEM((2, D, 128), jnp.float32),
        pltpu.SemaphoreType.DMA((2,)),
        pltpu.SemaphoreType.DMA((2,)),
    ],
    compiler_params=pltpu.CompilerParams(
        use_tc_tiling_on_sc=True, needs_layout_passes=False),
)
def _format_table(wt_hbm, tail_hbm, w2_hbm, cbuf, pbuf, sv, sw):
    wid = lax.axis_index("s") * NC + lax.axis_index("c")
    iota = lax.iota(jnp.int32, 16)
    dvecs = [D0 + iota for D0 in range(0, D, 16)]

    def fire_slab(i, buf):
        vt = wid + i * NW

        @pl.when(vt < NVT - 1)
        def _body():
            pltpu.async_copy(
                wt_hbm.at[:, pl.ds(vt * 128, 128)], cbuf.at[buf], sv.at[buf])

        @pl.when(vt == NVT - 1)
        def _tail():
            pltpu.async_copy(tail_hbm, cbuf.at[buf], sv.at[buf])

    def transpose(buf):
        @plsc.parallel_loop(0, 16, unroll=2)
        def _j(j):
            perm = lax.rem(iota + j, 16)
            rbase = perm // 2
            cvecs = [lax.rem(perm, 2) * 64 + dv for dv in dvecs]
            vvecs = [V0 + perm for V0 in range(0, 128, 16)]
            rvecs = [V0 // 2 + rbase for V0 in range(0, 128, 16)]
            for di in range(4):
                for vi in range(8):
                    x = plsc.load_gather(cbuf.at[buf], [dvecs[di], vvecs[vi]])
                    plsc.store_scatter(
                        pbuf.at[buf], [rvecs[vi], cvecs[di]], x)

    def fire_write(vt, buf):
        @pl.when(vt < NVT - 1)
        def _full():
            pltpu.async_copy(pbuf.at[buf],
                             w2_hbm.at[pl.ds(vt * 64, 64)], sw.at[buf])

        @pl.when(vt == NVT - 1)
        def _tail():
            pltpu.async_copy(pbuf.at[buf, pl.ds(0, 32)],
                             w2_hbm.at[pl.ds(vt * 64, 32)], sw.at[buf])

    def wait_write(vt, buf):
        @pl.when(vt < NVT - 1)
        def _full():
            pltpu.make_async_copy(
                pbuf.at[buf], w2_hbm.at[pl.ds(0, 64)], sw.at[buf]).wait()

        @pl.when(vt == NVT - 1)
        def _tail():
            pltpu.make_async_copy(
                pbuf.at[buf, pl.ds(0, 32)],
                w2_hbm.at[pl.ds(0, 32)], sw.at[buf]).wait()

    @pl.loop(0, NVT_W)
    def _slab(i):
        vt = wid + i * NW

        @pl.when(vt < NVT)
        def _active():
            buf = lax.rem(i, 2)

            @pl.when(i == 0)
            def _prime():
                fire_slab(i, buf)

            pltpu.make_async_copy(
                wt_hbm.at[:, pl.ds(0, 128)], cbuf.at[buf], sv.at[buf]).wait()

            @pl.when(wid + (i + 1) * NW < NVT)
            def _next():
                fire_slab(i + 1, 1 - buf)

            @pl.when(i >= 2)
            def _drain():
                wait_write(wid + (i - 2) * NW, buf)

            transpose(buf)
            fire_write(vt, buf)

    n_act = jnp.where(wid < NVT - (NVT_W - 1) * NW, NVT_W, NVT_W - 1)

    @pl.loop(0, 2)
    def _final(q):
        i_last = n_act - 1 - q
        wait_write(wid + i_last * NW, lax.rem(i_last, 2))


@functools.partial(
    pl.kernel,
    out_type=jax.ShapeDtypeStruct((S, 4, 2, NBT, 8, 128), jnp.float32),
    mesh=_mesh,
    scratch_types=[
        pltpu.VMEM((S, BTW, 128), jnp.int32),
        pltpu.VMEM((2, 128, D), jnp.float32),
        pltpu.VMEM((2, 4, 2, 8, 131), jnp.float32),
        pltpu.SemaphoreType.DMA((2,)),
        pltpu.SemaphoreType.DMA((2,)),
    ],
    compiler_params=pltpu.CompilerParams(
        use_tc_tiling_on_sc=False, needs_layout_passes=False),
)
def _emb_lookup(idx_hbm, table_hbm, out_hbm, idx_v, gbuf, tbuf, sg, sw):
    wid = lax.axis_index("s") * NC + lax.axis_index("c")
    bt0 = wid * BTW
    pltpu.sync_copy(idx_hbm.at[:, pl.ds(bt0, BTW)], idx_v)

    iota = lax.iota(jnp.int32, 16)
    dtlo_c = iota // 8
    dr_c = lax.rem(iota, 8)

    def fire_gather(p, par):
        s = p // BTW
        bl = lax.rem(p, BTW)
        pltpu.async_copy(
            table_hbm.at[idx_v.at[s, bl]], gbuf.at[par], sg.at[par])

    def transpose(par):
        @plsc.parallel_loop(0, 128, unroll=8)
        def _bc(bc):
            bcv = jnp.zeros((16,), jnp.int32) + bc
            for g in range(4):
                x = gbuf[par, bc, pl.ds(g * 16, 16)]
                plsc.store_scatter(tbuf.at[par, g], [dtlo_c, dr_c, bcv], x)

    @pl.loop(0, NP)
    def _pair(p):
        par = lax.rem(p, 2)
        s = p // BTW
        bl = lax.rem(p, BTW)

        @pl.when(p == 0)
        def _prime():
            fire_gather(p, par)

        pltpu.make_async_copy(
            table_hbm.at[idx_v.at[0, 0]], gbuf.at[par], sg.at[par]).wait()

        @pl.when(p < NP - 1)
        def _next():
            fire_gather(p + 1, 1 - par)

        @pl.when(p >= 2)
        def _drain():
            pltpu.make_async_copy(
                tbuf.at[par, :, :, :, pl.ds(0, 128)],
                out_hbm.at[0, :, :, 0], sw.at[par]).wait()

        transpose(par)
        pltpu.async_copy(
            tbuf.at[par, :, :, :, pl.ds(0, 128)],
            out_hbm.at[s, :, :, bt0 + bl], sw.at[par])

    for q in range(2):
        pltpu.make_async_copy(
            tbuf.at[q, :, :, :, pl.ds(0, 128)],
            out_hbm.at[0, :, :, 0], sw.at[q]).wait()


def kernel(input, weight):
    idx3 = input.T.reshape(S, NBT, 128).astype(jnp.int32)
    wt = weight.T
    tailp = jnp.pad(wt[:, (NVT - 1) * 128:], ((0, 0), (0, 64)))
    table = _format_table(wt, tailp).reshape(1000000, D)
    out5 = _emb_lookup(idx3, table).reshape(S, 8, NBT, 8, 128)
    return out5.transpose(2, 4, 0, 1, 3).reshape(16384, S, D)
